# baseline (speedup 1.0000x reference)
import functools

import jax
import jax.numpy as jnp
from jax import lax
from jax.experimental import pallas as pl
from jax.experimental.pallas import tpu as pltpu
from jax.experimental.pallas import tpu_sc as plsc


def _lane_broadcast(v16, lane):
    idx = jnp.full((16, 1), lane, jnp.int32)
    dnums = lax.GatherDimensionNumbers(
        offset_dims=(), collapsed_slice_dims=(0,), start_index_map=(0,))
    return lax.gather(v16, idx, dnums, slice_sizes=(1,),
                      mode=lax.GatherScatterMode.PROMISE_IN_BOUNDS)


def _make_sc_partials(n, e, d, nc, ns):
    nw = nc * ns
    epw = e // nw
    K = 80
    nt = epw // K
    assert epw % K == 0 and e % nw == 0
    ZR = 32
    rpt = ((n + ns * ZR - 1) // (ns * ZR)) * ZR
    n_pad = rpt * ns
    assert nt % 3 == 2 and nt >= 5

    mesh = plsc.VectorSubcoreMesh(core_axis_name="c", subcore_axis_name="s")

    @functools.partial(
        pl.kernel,
        mesh=mesh,
        out_type=jax.ShapeDtypeStruct((nc, n_pad, d), jnp.float32),
        scratch_types=(
            [pltpu.VMEM((K,), jnp.int32) for _ in range(3)]
            + [pltpu.VMEM((K,), jnp.int32) for _ in range(3)]
            + [pltpu.VMEM((K,), jnp.float32) for _ in range(3)]
            + [pltpu.VMEM((K, d), jnp.float32) for _ in range(3)]
            + [pltpu.VMEM((ZR, d), jnp.float32)]
            + [pltpu.VMEM_SHARED((n_pad, d), jnp.float32)]
            + [pltpu.SemaphoreType.DMA for _ in range(12)]
        ),
    )
    def sc_kernel(x_h, src_h, dst_h, ew_h, out_h, *refs):
        cid = lax.axis_index("c")
        sid = lax.axis_index("s")
        wid = sid * nc + cid
        src_v = refs[0:3]
        dst_v = refs[3:6]
        w_v = refs[6:9]
        rows_v = refs[9:12]
        zbuf = refs[12]
        acc = refs[13]
        csem = refs[14:17]
        wdsem = refs[17:20]
        gsem = refs[20:23]
        ssem = refs[23:26]

        def start_src(c, s):
            base = wid * epw + c * K
            pltpu.async_copy(src_h.at[pl.ds(base, K)], src_v[s], csem[s])

        def wait_src(s):
            pltpu.make_async_copy(src_h.at[pl.ds(0, K)], src_v[s],
                                  csem[s]).wait()

        def start_wd(c, s):
            base = wid * epw + c * K
            pltpu.async_copy(ew_h.at[pl.ds(base, K)], w_v[s], wdsem[s])
            pltpu.async_copy(dst_h.at[pl.ds(base, K)], dst_v[s], wdsem[s])

        def wait_wd(s):
            z = pl.ds(0, K)
            pltpu.make_async_copy(ew_h.at[z], w_v[s], wdsem[s]).wait()
            pltpu.make_async_copy(dst_h.at[z], dst_v[s], wdsem[s]).wait()

        def start_gather(s):
            pltpu.async_copy(x_h.at[src_v[s]], rows_v[s], gsem[s])

        def wait_gather(s):
            pltpu.make_async_copy(x_h.at[src_v[s]], rows_v[s],
                                  gsem[s]).wait()

        def start_scatter(s):
            pltpu.async_copy(rows_v[s], acc.at[dst_v[s]], ssem[s], add=True)

        def wait_scatter(s):
            pltpu.make_async_copy(rows_v[s], acc.at[dst_v[s]],
                                  ssem[s]).wait()

        def multiply(s):
            @plsc.parallel_loop(0, K // 16, unroll=4)
            def group(g):
                w16 = w_v[s][pl.ds(g * 16, 16)]
                for i in range(16):
                    wbc = _lane_broadcast(w16, i)
                    ei = g * 16 + i
                    for cg in range(d // 16):
                        sl = pl.ds(cg * 16, 16)
                        rows_v[s][ei, sl] = rows_v[s][ei, sl] * wbc

        start_src(0, 0)
        start_wd(0, 0)
        start_src(1, 1)
        start_wd(1, 1)
        wait_src(0)
        start_gather(0)
        wait_src(1)
        start_gather(1)

        zero16 = jnp.zeros((16,), jnp.float32)

        def zrow(r, carry):
            for cg in range(d // 16):
                zbuf[r, pl.ds(cg * 16, 16)] = zero16
            return carry

        lax.fori_loop(0, ZR, zrow, 0)
        for j in range(rpt // ZR):
            pltpu.sync_copy(zbuf, acc.at[pl.ds(sid * rpt + j * ZR, ZR)])
        plsc.subcore_barrier()

        def phase(c, s, guard=None):
            s1 = (s + 2) % 3
            wait_gather(s)
            start_src(c + 2, s1)
            wait_wd(s)
            multiply(s)
            start_scatter(s)
            if guard is None:
                wait_scatter(s1)
            else:
                @pl.when(guard)
                def _():
                    wait_scatter(s1)
            start_wd(c + 2, s1)
            wait_src(s1)
            start_gather(s1)

        def triple(u, carry):
            c0 = 3 * u
            phase(c0, 0, guard=u > 0)
            phase(c0 + 1, 1)
            phase(c0 + 2, 2)
            return carry

        lax.fori_loop(0, (nt - 2) // 3, triple, 0)
        for s in (0, 1):
            wait_gather(s)
            wait_wd(s)
            multiply(s)
            start_scatter(s)
        for s in range(3):
            wait_scatter(s)
        plsc.subcore_barrier()

        pltpu.sync_copy(acc.at[pl.ds(sid * rpt, rpt)],
                        out_h.at[cid, pl.ds(sid * rpt, rpt)])

    return sc_kernel


def _combine(partials, w2d, n):
    nc, _, d = partials.shape
    blk = 1000

    def body(p_ref, w_ref, o_ref):
        o_ref[...] = (p_ref[0] + p_ref[1]) * w_ref[...]

    return pl.pallas_call(
        body,
        grid=(n // blk,),
        in_specs=[
            pl.BlockSpec((nc, blk, d), lambda i: (0, i, 0)),
            pl.BlockSpec((1, d), lambda i: (0, 0)),
        ],
        out_specs=pl.BlockSpec((blk, d), lambda i: (i, 0)),
        out_shape=jax.ShapeDtypeStruct((n, d), jnp.float32),
    )(partials, w2d)


def kernel(x, edge_index, edge_weight, W):
    n, d = x.shape
    e = edge_index.shape[1]
    info = plsc.get_sparse_core_info()
    nc, ns = info.num_cores, info.num_subcores
    dst = edge_index[0]
    src = edge_index[1]
    partials = _make_sc_partials(n, e, d, nc, ns)(x, src, dst, edge_weight)
    return _combine(partials, W.reshape(1, d), n)


# speedup vs baseline: 1.1072x; 1.1072x over previous
import functools

import jax
import jax.numpy as jnp
from jax import lax
from jax.experimental import pallas as pl
from jax.experimental.pallas import tpu as pltpu
from jax.experimental.pallas import tpu_sc as plsc


def _lane_broadcast(v16, lane):
    idx = jnp.full((16, 1), lane, jnp.int32)
    dnums = lax.GatherDimensionNumbers(
        offset_dims=(), collapsed_slice_dims=(0,), start_index_map=(0,))
    return lax.gather(v16, idx, dnums, slice_sizes=(1,),
                      mode=lax.GatherScatterMode.PROMISE_IN_BOUNDS)


def _make_sc_partials(n, e, d, nc, ns):
    nw = nc * ns
    epw = e // nw
    K = 80
    nt = epw // K
    assert epw % K == 0 and e % nw == 0
    ZR = 32
    rpt = ((n + ns * ZR - 1) // (ns * ZR)) * ZR
    n_pad = rpt * ns
    assert nt % 3 == 2 and nt >= 5

    mesh = plsc.VectorSubcoreMesh(core_axis_name="c", subcore_axis_name="s")

    @functools.partial(
        pl.kernel,
        mesh=mesh,
        out_type=jax.ShapeDtypeStruct((nc, n_pad, d), jnp.float32),
        scratch_types=(
            [pltpu.VMEM((K,), jnp.int32) for _ in range(3)]
            + [pltpu.VMEM((K,), jnp.int32) for _ in range(3)]
            + [pltpu.VMEM((K,), jnp.float32) for _ in range(3)]
            + [pltpu.VMEM((K, d), jnp.float32) for _ in range(3)]
            + [pltpu.VMEM((ZR, d), jnp.float32)]
            + [pltpu.VMEM_SHARED((n_pad, d), jnp.float32)]
            + [pltpu.SemaphoreType.DMA for _ in range(12)]
        ),
    )
    def sc_kernel(x_h, src_h, dst_h, ew_h, out_h, *refs):
        cid = lax.axis_index("c")
        sid = lax.axis_index("s")
        wid = sid * nc + cid
        src_v = refs[0:3]
        dst_v = refs[3:6]
        w_v = refs[6:9]
        rows_v = refs[9:12]
        zbuf = refs[12]
        acc = refs[13]
        csem = refs[14:17]
        wdsem = refs[17:20]
        gsem = refs[20:23]
        ssem = refs[23:26]

        def start_src(c, s):
            base = wid * epw + c * K
            pltpu.async_copy(src_h.at[pl.ds(base, K)], src_v[s], csem[s])

        def wait_src(s):
            pltpu.make_async_copy(src_h.at[pl.ds(0, K)], src_v[s],
                                  csem[s]).wait()

        def start_wd(c, s):
            base = wid * epw + c * K
            pltpu.async_copy(ew_h.at[pl.ds(base, K)], w_v[s], wdsem[s])
            pltpu.async_copy(dst_h.at[pl.ds(base, K)], dst_v[s], wdsem[s])

        def wait_wd(s):
            z = pl.ds(0, K)
            pltpu.make_async_copy(ew_h.at[z], w_v[s], wdsem[s]).wait()
            pltpu.make_async_copy(dst_h.at[z], dst_v[s], wdsem[s]).wait()

        def start_gather(s):
            pltpu.async_copy(x_h.at[src_v[s]], rows_v[s], gsem[s])

        def wait_gather(s):
            pltpu.make_async_copy(x_h.at[src_v[s]], rows_v[s],
                                  gsem[s]).wait()

        def start_scatter(s):
            pltpu.async_copy(rows_v[s], acc.at[dst_v[s]], ssem[s], add=True)

        def wait_scatter(s):
            pltpu.make_async_copy(rows_v[s], acc.at[dst_v[s]],
                                  ssem[s]).wait()

        def multiply(s):
            @plsc.parallel_loop(0, K // 16, unroll=1)
            def group(g):
                w16 = w_v[s][pl.ds(g * 16, 16)]
                for i in range(16):
                    wbc = _lane_broadcast(w16, i)
                    ei = g * 16 + i
                    for cg in range(d // 16):
                        sl = pl.ds(cg * 16, 16)
                        rows_v[s][ei, sl] = rows_v[s][ei, sl] * wbc

        start_src(0, 0)
        start_wd(0, 0)
        start_src(1, 1)
        start_wd(1, 1)
        wait_src(0)
        start_gather(0)
        wait_src(1)
        start_gather(1)

        zero16 = jnp.zeros((16,), jnp.float32)

        def zrow(r, carry):
            for cg in range(d // 16):
                zbuf[r, pl.ds(cg * 16, 16)] = zero16
            return carry

        lax.fori_loop(0, ZR, zrow, 0)
        for j in range(rpt // ZR):
            pltpu.sync_copy(zbuf, acc.at[pl.ds(sid * rpt + j * ZR, ZR)])
        plsc.subcore_barrier()

        def phase(c, s, guard=None):
            s1 = (s + 2) % 3
            wait_gather(s)
            start_src(c + 2, s1)
            wait_wd(s)
            multiply(s)
            start_scatter(s)
            if guard is None:
                wait_scatter(s1)
            else:
                @pl.when(guard)
                def _():
                    wait_scatter(s1)
            start_wd(c + 2, s1)
            wait_src(s1)
            start_gather(s1)

        def triple(u, carry):
            c0 = 3 * u
            phase(c0, 0, guard=u > 0)
            phase(c0 + 1, 1)
            phase(c0 + 2, 2)
            return carry

        lax.fori_loop(0, (nt - 2) // 3, triple, 0)
        for s in (0, 1):
            wait_gather(s)
            wait_wd(s)
            multiply(s)
            start_scatter(s)
        for s in range(3):
            wait_scatter(s)
        plsc.subcore_barrier()

        pltpu.sync_copy(acc.at[pl.ds(sid * rpt, rpt)],
                        out_h.at[cid, pl.ds(sid * rpt, rpt)])

    return sc_kernel


def _combine(partials, w2d, n):
    nc, _, d = partials.shape
    blk = 1000

    def body(p_ref, w_ref, o_ref):
        o_ref[...] = (p_ref[0] + p_ref[1]) * w_ref[...]

    return pl.pallas_call(
        body,
        grid=(n // blk,),
        in_specs=[
            pl.BlockSpec((nc, blk, d), lambda i: (0, i, 0)),
            pl.BlockSpec((1, d), lambda i: (0, 0)),
        ],
        out_specs=pl.BlockSpec((blk, d), lambda i: (i, 0)),
        out_shape=jax.ShapeDtypeStruct((n, d), jnp.float32),
    )(partials, w2d)


def kernel(x, edge_index, edge_weight, W):
    n, d = x.shape
    e = edge_index.shape[1]
    info = plsc.get_sparse_core_info()
    nc, ns = info.num_cores, info.num_subcores
    dst = edge_index[0]
    src = edge_index[1]
    partials = _make_sc_partials(n, e, d, nc, ns)(x, src, dst, edge_weight)
    return _combine(partials, W.reshape(1, d), n)


# speedup vs baseline: 1.2249x; 1.1063x over previous
import functools

import jax
import jax.numpy as jnp
from jax import lax
from jax.experimental import pallas as pl
from jax.experimental.pallas import tpu as pltpu
from jax.experimental.pallas import tpu_sc as plsc


def _lane_broadcast(v16, lane):
    idx = jnp.full((16, 1), lane, jnp.int32)
    dnums = lax.GatherDimensionNumbers(
        offset_dims=(), collapsed_slice_dims=(0,), start_index_map=(0,))
    return lax.gather(v16, idx, dnums, slice_sizes=(1,),
                      mode=lax.GatherScatterMode.PROMISE_IN_BOUNDS)


def _make_sc_partials(n, e, d, nc, ns):
    nw = nc * ns
    epw = e // nw
    K = 80
    nt = epw // K
    assert epw % K == 0 and e % nw == 0
    ZR = 32
    rpt = ((n + ns * ZR - 1) // (ns * ZR)) * ZR
    n_pad = rpt * ns
    assert nt % 3 == 2 and nt >= 5

    mesh = plsc.VectorSubcoreMesh(core_axis_name="c", subcore_axis_name="s")

    @functools.partial(
        pl.kernel,
        mesh=mesh,
        out_type=jax.ShapeDtypeStruct((nc, n_pad, d), jnp.float32),
        scratch_types=(
            [pltpu.VMEM((K,), jnp.int32) for _ in range(3)]
            + [pltpu.VMEM((K,), jnp.int32) for _ in range(3)]
            + [pltpu.VMEM((K + 16,), jnp.float32) for _ in range(3)]
            + [pltpu.VMEM((K, d), jnp.float32) for _ in range(3)]
            + [pltpu.VMEM((ZR, d), jnp.float32)]
            + [pltpu.VMEM_SHARED((n_pad, d), jnp.float32)]
            + [pltpu.SemaphoreType.DMA for _ in range(12)]
        ),
    )
    def sc_kernel(x_h, src_h, dst_h, ew_h, out_h, *refs):
        cid = lax.axis_index("c")
        sid = lax.axis_index("s")
        wid = sid * nc + cid
        src_v = refs[0:3]
        dst_v = refs[3:6]
        w_v = refs[6:9]
        rows_v = refs[9:12]
        zbuf = refs[12]
        acc = refs[13]
        csem = refs[14:17]
        wdsem = refs[17:20]
        gsem = refs[20:23]
        ssem = refs[23:26]

        def start_src(c, s):
            base = wid * epw + c * K
            pltpu.async_copy(src_h.at[pl.ds(base, K)], src_v[s], csem[s])

        def wait_src(s):
            pltpu.make_async_copy(src_h.at[pl.ds(0, K)], src_v[s],
                                  csem[s]).wait()

        def start_wd(c, s):
            base = wid * epw + c * K
            pltpu.async_copy(ew_h.at[pl.ds(base, K)], w_v[s].at[pl.ds(0, K)], wdsem[s])
            pltpu.async_copy(dst_h.at[pl.ds(base, K)], dst_v[s], wdsem[s])

        def wait_wd(s):
            z = pl.ds(0, K)
            pltpu.make_async_copy(ew_h.at[z], w_v[s].at[pl.ds(0, K)], wdsem[s]).wait()
            pltpu.make_async_copy(dst_h.at[z], dst_v[s], wdsem[s]).wait()

        def start_gather(s):
            pltpu.async_copy(x_h.at[src_v[s]], rows_v[s], gsem[s])

        def wait_gather(s):
            pltpu.make_async_copy(x_h.at[src_v[s]], rows_v[s],
                                  gsem[s]).wait()

        def start_scatter(s):
            pltpu.async_copy(rows_v[s], acc.at[dst_v[s]], ssem[s], add=True)

        def wait_scatter(s):
            pltpu.make_async_copy(rows_v[s], acc.at[dst_v[s]],
                                  ssem[s]).wait()

        def multiply(s):
            @plsc.parallel_loop(0, K, unroll=4)
            def edge(ei):
                w16 = w_v[s][pl.ds(ei, 16)]
                wbc = _lane_broadcast(w16, 0)
                for cg in range(d // 16):
                    sl = pl.ds(cg * 16, 16)
                    rows_v[s][ei, sl] = rows_v[s][ei, sl] * wbc

        start_src(0, 0)
        start_wd(0, 0)
        start_src(1, 1)
        start_wd(1, 1)
        wait_src(0)
        start_gather(0)
        wait_src(1)
        start_gather(1)

        zero16 = jnp.zeros((16,), jnp.float32)

        def zrow(r, carry):
            for cg in range(d // 16):
                zbuf[r, pl.ds(cg * 16, 16)] = zero16
            return carry

        lax.fori_loop(0, ZR, zrow, 0)
        for j in range(rpt // ZR):
            pltpu.sync_copy(zbuf, acc.at[pl.ds(sid * rpt + j * ZR, ZR)])
        plsc.subcore_barrier()

        def phase(c, s, guard=None):
            s1 = (s + 2) % 3
            wait_gather(s)
            start_src(c + 2, s1)
            wait_wd(s)
            multiply(s)
            start_scatter(s)
            if guard is None:
                wait_scatter(s1)
            else:
                @pl.when(guard)
                def _():
                    wait_scatter(s1)
            start_wd(c + 2, s1)
            wait_src(s1)
            start_gather(s1)

        def triple(u, carry):
            c0 = 3 * u
            phase(c0, 0, guard=u > 0)
            phase(c0 + 1, 1)
            phase(c0 + 2, 2)
            return carry

        lax.fori_loop(0, (nt - 2) // 3, triple, 0)
        for s in (0, 1):
            wait_gather(s)
            wait_wd(s)
            multiply(s)
            start_scatter(s)
        for s in range(3):
            wait_scatter(s)
        plsc.subcore_barrier()

        pltpu.sync_copy(acc.at[pl.ds(sid * rpt, rpt)],
                        out_h.at[cid, pl.ds(sid * rpt, rpt)])

    return sc_kernel


def _combine(partials, w2d, n):
    nc, _, d = partials.shape
    blk = 1000

    def body(p_ref, w_ref, o_ref):
        o_ref[...] = (p_ref[0] + p_ref[1]) * w_ref[...]

    return pl.pallas_call(
        body,
        grid=(n // blk,),
        in_specs=[
            pl.BlockSpec((nc, blk, d), lambda i: (0, i, 0)),
            pl.BlockSpec((1, d), lambda i: (0, 0)),
        ],
        out_specs=pl.BlockSpec((blk, d), lambda i: (i, 0)),
        out_shape=jax.ShapeDtypeStruct((n, d), jnp.float32),
    )(partials, w2d)


def kernel(x, edge_index, edge_weight, W):
    n, d = x.shape
    e = edge_index.shape[1]
    info = plsc.get_sparse_core_info()
    nc, ns = info.num_cores, info.num_subcores
    dst = edge_index[0]
    src = edge_index[1]
    partials = _make_sc_partials(n, e, d, nc, ns)(x, src, dst, edge_weight)
    return _combine(partials, W.reshape(1, d), n)
